# Initial kernel scaffold; baseline (speedup 1.0000x reference)
import jax
import jax.numpy as jnp
from jax.experimental import pallas as pl


def kernel(entity_table, relation_table, h_idx, r_idx, t_idx, y):
    raise NotImplementedError("write your pallas kernel here")


# speedup vs baseline: 1.5845x; 1.5845x over previous
import jax
import jax.numpy as jnp
from jax import lax
from jax.experimental import pallas as pl
from jax.experimental.pallas import tpu as pltpu, tpu_sc as plsc

_NC, _NS = 2, 16
_NW = _NC * _NS
_B = 16384
_DIM = 64
_BPW = _B // _NW
_CHUNK = 128
_NCH = _BPW // _CHUNK


def _gather_body(ent_hbm, rel_hbm, h_hbm, r_hbm, t_hbm, out_hbm,
                 hidx, ridx, tidx, hbuf, rbuf, tbuf, sem):
    wid = lax.axis_index("s") * _NC + lax.axis_index("c")
    base = wid * _BPW
    pltpu.sync_copy(h_hbm.at[pl.ds(base, _BPW)], hidx)
    pltpu.sync_copy(r_hbm.at[pl.ds(base, _BPW)], ridx)
    pltpu.sync_copy(t_hbm.at[pl.ds(base, _BPW)], tidx)
    copies = []
    for j in range(_NCH):
        s = pl.ds(j * _CHUNK, _CHUNK)
        copies.append(pltpu.async_copy(ent_hbm.at[hidx.at[s]], hbuf.at[s], sem))
        copies.append(pltpu.async_copy(rel_hbm.at[ridx.at[s]], rbuf.at[s], sem))
        copies.append(pltpu.async_copy(ent_hbm.at[tidx.at[s]], tbuf.at[s], sem))
    for c in copies:
        c.wait()
    rows = pl.ds(base, _BPW)
    pltpu.sync_copy(hbuf, out_hbm.at[rows, pl.ds(0, _DIM)])
    pltpu.sync_copy(rbuf, out_hbm.at[rows, pl.ds(_DIM, _DIM)])
    pltpu.sync_copy(tbuf, out_hbm.at[rows, pl.ds(2 * _DIM, _DIM)])


def kernel(entity_table, relation_table, h_idx, r_idx, t_idx, y):
    mesh = plsc.VectorSubcoreMesh(core_axis_name="c", subcore_axis_name="s")
    out = pl.kernel(
        _gather_body,
        out_type=jax.ShapeDtypeStruct((_B, 3 * _DIM), jnp.float32),
        mesh=mesh,
        compiler_params=pltpu.CompilerParams(use_tc_tiling_on_sc=False),
        scratch_types=[
            pltpu.VMEM((_BPW,), jnp.int32),
            pltpu.VMEM((_BPW,), jnp.int32),
            pltpu.VMEM((_BPW,), jnp.int32),
            pltpu.VMEM((_BPW, _DIM), jnp.float32),
            pltpu.VMEM((_BPW, _DIM), jnp.float32),
            pltpu.VMEM((_BPW, _DIM), jnp.float32),
            pltpu.SemaphoreType.DMA,
        ],
    )(entity_table, relation_table,
      h_idx.astype(jnp.int32), r_idx.astype(jnp.int32), t_idx.astype(jnp.int32))
    return out.reshape(-1)
